# baseline (speedup 1.0000x reference)
import jax
import jax.numpy as jnp
from jax import lax
from jax.experimental import pallas as pl
from jax.experimental.pallas import tpu as pltpu
from jax.experimental.pallas import tpu_sc as plsc

_B = 4
_D = 256
_H = 32
_W = 32
_ROWS_PER_B = _H * _W


def _body(ce_hbm, re_hbm, out_hbm, ce_v, re_v, buf, sem):
    wid = lax.axis_index("s") * 2 + lax.axis_index("c")

    pltpu.sync_copy(ce_hbm.at[pl.ds(0, _W)], ce_v)
    pltpu.sync_copy(re_hbm.at[pl.ds(wid, 1)], re_v)

    rv = [re_v[0, pl.ds(k * 16, 16)] for k in range(_D // 16)]

    def fill(j, _):
        for k in range(_D // 16):
            buf[j, pl.ds(k * 16, 16)] = ce_v[j, pl.ds(k * 16, 16)]
        for k in range(_D // 16):
            buf[j, pl.ds(_D + k * 16, 16)] = rv[k]
        return 0

    lax.fori_loop(0, _W, fill, 0)

    for b in range(_B):
        pltpu.async_copy(buf, out_hbm.at[b, pl.ds(wid * _W, _W)], sem)
    for b in range(_B):
        pltpu.make_async_copy(buf, out_hbm.at[b, pl.ds(wid * _W, _W)], sem).wait()


@jax.jit
def _pos_sc(ce, re):
    mesh = plsc.VectorSubcoreMesh(core_axis_name="c", subcore_axis_name="s")
    pk = pl.kernel(
        _body,
        out_type=jax.ShapeDtypeStruct((_B, _ROWS_PER_B, 2 * _D), jnp.float32),
        mesh=mesh,
        scratch_types=[
            pltpu.VMEM((_H, _D), jnp.float32),
            pltpu.VMEM((1, _D), jnp.float32),
            pltpu.VMEM((_W, 2 * _D), jnp.float32),
            pltpu.SemaphoreType.DMA,
        ],
    )(ce, re)
    return jnp.transpose(pk.reshape(_B, _H, _W, 2 * _D), (0, 3, 1, 2))


def kernel(tensors, row_embed, col_embed):
    return _pos_sc(col_embed, row_embed)


# speedup vs baseline: 1.1612x; 1.1612x over previous
import jax
import jax.numpy as jnp
from jax import lax
from jax.experimental import pallas as pl
from jax.experimental.pallas import tpu as pltpu
from jax.experimental.pallas import tpu_sc as plsc

_B = 4
_D = 256
_H = 32
_W = 32
_ROWS_PER_B = _H * _W


def _body(ce_hbm, re_hbm, out_hbm, ce_v, re_v, buf, sem):
    wid = lax.axis_index("s") * 2 + lax.axis_index("c")

    pltpu.sync_copy(ce_hbm.at[pl.ds(0, _W)], ce_v)
    pltpu.sync_copy(re_hbm.at[pl.ds(wid, 1)], re_v)

    rv = [re_v[0, pl.ds(k * 16, 16)] for k in range(_D // 16)]

    def fill(j, _):
        for k in range(_D // 16):
            buf[j, pl.ds(k * 16, 16)] = rv[k]
        return 0

    for b in range(_B):
        pltpu.async_copy(
            ce_v, out_hbm.at[b, pl.ds(wid * _W, _W), pl.ds(0, _D)], sem
        )
    lax.fori_loop(0, _W, fill, 0)
    for b in range(_B):
        pltpu.async_copy(
            buf, out_hbm.at[b, pl.ds(wid * _W, _W), pl.ds(_D, _D)], sem
        )
    for b in range(_B):
        pltpu.make_async_copy(
            ce_v, out_hbm.at[b, pl.ds(wid * _W, _W), pl.ds(0, _D)], sem
        ).wait()
        pltpu.make_async_copy(
            buf, out_hbm.at[b, pl.ds(wid * _W, _W), pl.ds(_D, _D)], sem
        ).wait()


@jax.jit
def _pos_sc(ce, re):
    mesh = plsc.VectorSubcoreMesh(core_axis_name="c", subcore_axis_name="s")
    pk = pl.kernel(
        _body,
        out_type=jax.ShapeDtypeStruct((_B, _ROWS_PER_B, 2 * _D), jnp.float32),
        mesh=mesh,
        scratch_types=[
            pltpu.VMEM((_H, _D), jnp.float32),
            pltpu.VMEM((1, _D), jnp.float32),
            pltpu.VMEM((_W, _D), jnp.float32),
            pltpu.SemaphoreType.DMA,
        ],
    )(ce, re)
    return jnp.transpose(pk.reshape(_B, _H, _W, 2 * _D), (0, 3, 1, 2))


def kernel(tensors, row_embed, col_embed):
    return _pos_sc(col_embed, row_embed)
